# Initial kernel scaffold; baseline (speedup 1.0000x reference)
import jax
import jax.numpy as jnp
from jax.experimental import pallas as pl


def kernel(x, adj, xdeg, ydeg, Wa1, ba1, Wa2, ba2, W1, Wl2, bl2, W2):
    raise NotImplementedError("write your pallas kernel here")


# speedup vs baseline: 1.8215x; 1.8215x over previous
import jax
import jax.numpy as jnp
from jax.experimental import pallas as pl

_N = 4096
_BR = 128
_CCH = 1024
_BM = 512


def _edge_kernel(params_ref, adj_ref, xdeg_ref, ydeg_ref,
                 a16_ref, rowsum_ref, colsum_ref):
    i = pl.program_id(0)

    @pl.when(i == 0)
    def _init():
        colsum_ref[...] = jnp.zeros_like(colsum_ref)

    p = params_ref[...]
    row0 = i * _BR

    for ci in range(_N // _CCH):
        c0 = ci * _CCH

        def body(ri, _):
            rs = pl.ds(ri * 8, 8)
            a = adj_ref[rs, c0:c0 + _CCH]
            xd = xdeg_ref[rs, c0:c0 + _CCH]
            yd = ydeg_ref[rs, c0:c0 + _CCH]
            logit = jnp.full((8, _CCH), 0.0, dtype=jnp.float32) + p[0, 80]
            for k in range(16):
                pk = (a * p[0, k] + xd * p[0, 16 + k]
                      + yd * p[0, 32 + k] + p[0, 48 + k])
                logit = logit + p[0, 64 + k] * jnp.maximum(pk, 0.0)
            mask = 1.0 / (1.0 + jnp.exp(-logit))
            mask = jnp.where(a != 0.0, mask, 0.0)
            av = a * mask
            rid = jax.lax.broadcasted_iota(jnp.int32, (8, _CCH), 0) \
                + (row0 + ri * 8)
            cid = jax.lax.broadcasted_iota(jnp.int32, (8, _CCH), 1) + c0
            av = jnp.where(rid == cid, av + 1.0, av)
            a16_ref[rs, c0:c0 + _CCH] = av.astype(jnp.bfloat16)
            part_r = jnp.sum(av, axis=1, keepdims=True)
            if ci == 0:
                rowsum_ref[rs, :] = part_r
            else:
                rowsum_ref[rs, :] = rowsum_ref[rs, :] + part_r
            colsum_ref[0:1, c0:c0 + _CCH] = (
                colsum_ref[0:1, c0:c0 + _CCH]
                + jnp.sum(av, axis=0, keepdims=True))
            return 0

        jax.lax.fori_loop(0, _BR // 8, body, 0)


def _prep_kernel(rowsum_ref, colsum_ref, x_ref, w1_ref,
                 drow_ref, dcol_ref, sd_ref):
    dr = jax.lax.rsqrt(rowsum_ref[...])
    dr = jnp.where(jnp.isinf(dr), 0.0, dr)
    drow_ref[...] = jnp.broadcast_to(dr, (_N, 128))
    dc = jax.lax.rsqrt(colsum_ref[...])
    dc = jnp.where(jnp.isinf(dc), 0.0, dc)
    dcol_ref[...] = jnp.broadcast_to(dc, (_N, 128))
    sup = jnp.dot(x_ref[...].astype(jnp.bfloat16), w1_ref[...],
                  preferred_element_type=jnp.float32)
    sd_ref[...] = (sup * dc).astype(jnp.bfloat16)


def _prop1_kernel(a16_ref, sd_ref, drow_ref, dcol_ref, wl2_ref, bl2_ref,
                  w2_ref, hid_ref, s2d_ref):
    out1 = jnp.dot(a16_ref[...], sd_ref[...],
                   preferred_element_type=jnp.float32)
    out1 = out1 * drow_ref[:, 0:1]
    hid = jnp.dot(out1.astype(jnp.bfloat16), wl2_ref[...],
                  preferred_element_type=jnp.float32) + bl2_ref[...]
    hid_ref[...] = hid
    s2 = jnp.dot(hid.astype(jnp.bfloat16), w2_ref[...],
                 preferred_element_type=jnp.float32)
    s2d_ref[...] = (s2 * dcol_ref[:, 0:1]).astype(jnp.bfloat16)


def _prop2_kernel(a16_ref, s2d_ref, drow_ref, out_ref):
    o = jnp.dot(a16_ref[...], s2d_ref[...],
                preferred_element_type=jnp.float32)
    out_ref[...] = o * drow_ref[:, 0:1]


def kernel(x, adj, xdeg, ydeg, Wa1, ba1, Wa2, ba2, W1, Wl2, bl2, W2):
    f32 = jnp.float32
    v2 = (Wa2[:, 1] - Wa2[:, 0]).astype(f32)
    c = (ba2[1] - ba2[0]).astype(f32)
    params = jnp.concatenate([
        Wa1[0].astype(f32), Wa1[1].astype(f32), Wa1[2].astype(f32),
        ba1.astype(f32), v2, jnp.full((16,), c, dtype=f32),
        jnp.zeros((32,), dtype=f32)]).reshape(1, 128)

    a16, rowsum, colsum = pl.pallas_call(
        _edge_kernel,
        grid=(_N // _BR,),
        in_specs=[
            pl.BlockSpec((1, 128), lambda i: (0, 0)),
            pl.BlockSpec((_BR, _N), lambda i: (i, 0)),
            pl.BlockSpec((_BR, _N), lambda i: (i, 0)),
            pl.BlockSpec((_BR, _N), lambda i: (i, 0)),
        ],
        out_specs=[
            pl.BlockSpec((_BR, _N), lambda i: (i, 0)),
            pl.BlockSpec((_BR, 1), lambda i: (i, 0)),
            pl.BlockSpec((1, _N), lambda i: (0, 0)),
        ],
        out_shape=[
            jax.ShapeDtypeStruct((_N, _N), jnp.bfloat16),
            jax.ShapeDtypeStruct((_N, 1), f32),
            jax.ShapeDtypeStruct((1, _N), f32),
        ],
    )(params, adj, xdeg, ydeg)

    colsum_col = colsum.reshape(_N, 1)
    drow, dcol, sd = pl.pallas_call(
        _prep_kernel,
        out_shape=[
            jax.ShapeDtypeStruct((_N, 128), f32),
            jax.ShapeDtypeStruct((_N, 128), f32),
            jax.ShapeDtypeStruct((_N, 256), jnp.bfloat16),
        ],
    )(rowsum, colsum_col, x, W1.astype(jnp.bfloat16))

    w2pad = jnp.zeros((64, 128), dtype=jnp.bfloat16).at[:, :2].set(
        W2.astype(jnp.bfloat16))
    hid, s2d = pl.pallas_call(
        _prop1_kernel,
        grid=(_N // _BM,),
        in_specs=[
            pl.BlockSpec((_BM, _N), lambda i: (i, 0)),
            pl.BlockSpec((_N, 256), lambda i: (0, 0)),
            pl.BlockSpec((_BM, 128), lambda i: (i, 0)),
            pl.BlockSpec((_BM, 128), lambda i: (i, 0)),
            pl.BlockSpec((256, 64), lambda i: (0, 0)),
            pl.BlockSpec((1, 64), lambda i: (0, 0)),
            pl.BlockSpec((64, 128), lambda i: (0, 0)),
        ],
        out_specs=[
            pl.BlockSpec((_BM, 64), lambda i: (i, 0)),
            pl.BlockSpec((_BM, 128), lambda i: (i, 0)),
        ],
        out_shape=[
            jax.ShapeDtypeStruct((_N, 64), f32),
            jax.ShapeDtypeStruct((_N, 128), jnp.bfloat16),
        ],
    )(a16, sd, drow, dcol, Wl2.astype(jnp.bfloat16), bl2.reshape(1, 64),
      w2pad)

    outp = pl.pallas_call(
        _prop2_kernel,
        grid=(_N // _BM,),
        in_specs=[
            pl.BlockSpec((_BM, _N), lambda i: (i, 0)),
            pl.BlockSpec((_N, 128), lambda i: (0, 0)),
            pl.BlockSpec((_BM, 128), lambda i: (i, 0)),
        ],
        out_specs=pl.BlockSpec((_BM, 128), lambda i: (i, 0)),
        out_shape=jax.ShapeDtypeStruct((_N, 128), f32),
    )(a16, s2d, drow)

    output = outp[:, :2]
    return (output, hid, output)


# speedup vs baseline: 2.4394x; 1.3392x over previous
import jax
import jax.numpy as jnp
from jax.experimental import pallas as pl

_N = 4096
_BR = 128
_CCH = 1024
_BM = 512


def _edge_kernel(params_ref, adj_ref, xdeg_ref, ydeg_ref,
                 a16_ref, rowsum_ref, colsum_ref):
    i = pl.program_id(0)

    @pl.when(i == 0)
    def _init():
        colsum_ref[...] = jnp.zeros_like(colsum_ref)

    p = params_ref[...]
    row0 = i * _BR
    w0 = [p[0, k] for k in range(16)]
    w1 = [p[0, 16 + k] for k in range(16)]
    w2 = [p[0, 32 + k] for k in range(16)]
    bb = [p[0, 48 + k] for k in range(16)]
    v2 = [p[0, 64 + k] for k in range(16)]
    cc = p[0, 80]

    for ci in range(_N // _CCH):
        c0 = ci * _CCH

        def body(ri, _):
            rs = pl.ds(ri * 8, 8)
            a = adj_ref[rs, c0:c0 + _CCH]
            xd = xdeg_ref[rs, c0:c0 + _CCH]
            yd = ydeg_ref[rs, c0:c0 + _CCH]
            acc0 = jnp.full((8, _CCH), 0.0, dtype=jnp.float32) + cc
            acc1 = jnp.zeros((8, _CCH), dtype=jnp.float32)
            for k in range(16):
                pk = (a * w0[k] + xd * w1[k] + yd * w2[k] + bb[k])
                term = v2[k] * jnp.maximum(pk, 0.0)
                if k % 2 == 0:
                    acc0 = acc0 + term
                else:
                    acc1 = acc1 + term
            logit = acc0 + acc1
            mask = 1.0 / (1.0 + jnp.exp(-logit))
            mask = jnp.where(a != 0.0, mask, 0.0)
            av = a * mask
            rid = jax.lax.broadcasted_iota(jnp.int32, (8, _CCH), 0) \
                + (row0 + ri * 8)
            cid = jax.lax.broadcasted_iota(jnp.int32, (8, _CCH), 1) + c0
            av = jnp.where(rid == cid, av + 1.0, av)
            a16_ref[rs, c0:c0 + _CCH] = av.astype(jnp.bfloat16)
            part_r = jnp.sum(av, axis=1, keepdims=True)
            if ci == 0:
                rowsum_ref[rs, :] = part_r
            else:
                rowsum_ref[rs, :] = rowsum_ref[rs, :] + part_r
            colsum_ref[0:1, c0:c0 + _CCH] = (
                colsum_ref[0:1, c0:c0 + _CCH]
                + jnp.sum(av, axis=0, keepdims=True))
            return 0

        jax.lax.fori_loop(0, _BR // 8, body, 0)


def _prep_kernel(rowsum_ref, colsum_ref, x_ref, w1_ref,
                 drow_ref, dcol_ref, sd_ref):
    dr = jax.lax.rsqrt(rowsum_ref[...])
    dr = jnp.where(jnp.isinf(dr), 0.0, dr)
    drow_ref[...] = jnp.broadcast_to(dr, (_N, 128))
    dc = jax.lax.rsqrt(colsum_ref[...])
    dc = jnp.where(jnp.isinf(dc), 0.0, dc)
    dcol_ref[...] = jnp.broadcast_to(dc, (_N, 128))
    sup = jnp.dot(x_ref[...].astype(jnp.bfloat16), w1_ref[...],
                  preferred_element_type=jnp.float32)
    sd_ref[...] = (sup * dc).astype(jnp.bfloat16)


def _prop1_kernel(a16_ref, sd_ref, drow_ref, dcol_ref, wl2_ref, bl2_ref,
                  w2_ref, hid_ref, s2d_ref):
    out1 = jnp.dot(a16_ref[...], sd_ref[...],
                   preferred_element_type=jnp.float32)
    out1 = out1 * drow_ref[:, 0:1]
    hid = jnp.dot(out1.astype(jnp.bfloat16), wl2_ref[...],
                  preferred_element_type=jnp.float32) + bl2_ref[...]
    hid_ref[...] = hid
    s2 = jnp.dot(hid.astype(jnp.bfloat16), w2_ref[...],
                 preferred_element_type=jnp.float32)
    s2d_ref[...] = (s2 * dcol_ref[:, 0:1]).astype(jnp.bfloat16)


def _prop2_kernel(a16_ref, s2d_ref, drow_ref, out_ref):
    o = jnp.dot(a16_ref[...], s2d_ref[...],
                preferred_element_type=jnp.float32)
    out_ref[...] = o * drow_ref[:, 0:1]


def kernel(x, adj, xdeg, ydeg, Wa1, ba1, Wa2, ba2, W1, Wl2, bl2, W2):
    f32 = jnp.float32
    v2 = (Wa2[:, 1] - Wa2[:, 0]).astype(f32)
    c = (ba2[1] - ba2[0]).astype(f32)
    params = jnp.concatenate([
        Wa1[0].astype(f32), Wa1[1].astype(f32), Wa1[2].astype(f32),
        ba1.astype(f32), v2, jnp.full((16,), c, dtype=f32),
        jnp.zeros((32,), dtype=f32)]).reshape(1, 128)

    a16, rowsum, colsum = pl.pallas_call(
        _edge_kernel,
        grid=(_N // _BR,),
        in_specs=[
            pl.BlockSpec((1, 128), lambda i: (0, 0)),
            pl.BlockSpec((_BR, _N), lambda i: (i, 0)),
            pl.BlockSpec((_BR, _N), lambda i: (i, 0)),
            pl.BlockSpec((_BR, _N), lambda i: (i, 0)),
        ],
        out_specs=[
            pl.BlockSpec((_BR, _N), lambda i: (i, 0)),
            pl.BlockSpec((_BR, 1), lambda i: (i, 0)),
            pl.BlockSpec((1, _N), lambda i: (0, 0)),
        ],
        out_shape=[
            jax.ShapeDtypeStruct((_N, _N), jnp.bfloat16),
            jax.ShapeDtypeStruct((_N, 1), f32),
            jax.ShapeDtypeStruct((1, _N), f32),
        ],
    )(params, adj, xdeg, ydeg)

    colsum_col = colsum.reshape(_N, 1)
    drow, dcol, sd = pl.pallas_call(
        _prep_kernel,
        out_shape=[
            jax.ShapeDtypeStruct((_N, 128), f32),
            jax.ShapeDtypeStruct((_N, 128), f32),
            jax.ShapeDtypeStruct((_N, 256), jnp.bfloat16),
        ],
    )(rowsum, colsum_col, x, W1.astype(jnp.bfloat16))

    w2pad = jnp.zeros((64, 128), dtype=jnp.bfloat16).at[:, :2].set(
        W2.astype(jnp.bfloat16))
    hid, s2d = pl.pallas_call(
        _prop1_kernel,
        grid=(_N // _BM,),
        in_specs=[
            pl.BlockSpec((_BM, _N), lambda i: (i, 0)),
            pl.BlockSpec((_N, 256), lambda i: (0, 0)),
            pl.BlockSpec((_BM, 128), lambda i: (i, 0)),
            pl.BlockSpec((_BM, 128), lambda i: (i, 0)),
            pl.BlockSpec((256, 64), lambda i: (0, 0)),
            pl.BlockSpec((1, 64), lambda i: (0, 0)),
            pl.BlockSpec((64, 128), lambda i: (0, 0)),
        ],
        out_specs=[
            pl.BlockSpec((_BM, 64), lambda i: (i, 0)),
            pl.BlockSpec((_BM, 128), lambda i: (i, 0)),
        ],
        out_shape=[
            jax.ShapeDtypeStruct((_N, 64), f32),
            jax.ShapeDtypeStruct((_N, 128), jnp.bfloat16),
        ],
    )(a16, sd, drow, dcol, Wl2.astype(jnp.bfloat16), bl2.reshape(1, 64),
      w2pad)

    outp = pl.pallas_call(
        _prop2_kernel,
        grid=(_N // _BM,),
        in_specs=[
            pl.BlockSpec((_BM, _N), lambda i: (i, 0)),
            pl.BlockSpec((_N, 128), lambda i: (0, 0)),
            pl.BlockSpec((_BM, 128), lambda i: (i, 0)),
        ],
        out_specs=pl.BlockSpec((_BM, 128), lambda i: (i, 0)),
        out_shape=jax.ShapeDtypeStruct((_N, 128), f32),
    )(a16, s2d, drow)

    output = outp[:, :2]
    return (output, hid, output)


# speedup vs baseline: 3.4479x; 1.4134x over previous
import jax
import jax.numpy as jnp
from jax.experimental import pallas as pl

_N = 4096
_BR = 128
_CCH = 1024
_BM = 512


def _edge_kernel(params_ref, adj_ref, xdeg_ref, ydeg_ref,
                 a16_ref, rowsum_ref, colsum_ref):
    i = pl.program_id(0)

    @pl.when(i == 0)
    def _init():
        colsum_ref[...] = jnp.zeros_like(colsum_ref)

    p = params_ref[...]
    row0 = i * _BR
    w0 = [p[0, k] for k in range(16)]
    w1 = [p[0, 16 + k] for k in range(16)]
    w2 = [p[0, 32 + k] for k in range(16)]
    bb = [p[0, 48 + k] for k in range(16)]
    v2 = [p[0, 64 + k] for k in range(16)]
    cc = p[0, 80]

    for ri in range(_BR // 8):
        rs = slice(ri * 8, ri * 8 + 8)
        rpart = None
        for ci in range(_N // _CCH):
            c0 = ci * _CCH
            a = adj_ref[rs, c0:c0 + _CCH]
            xd = xdeg_ref[rs, c0:c0 + _CCH]
            yd = ydeg_ref[rs, c0:c0 + _CCH]
            acc0 = jnp.full((8, _CCH), 0.0, dtype=jnp.float32) + cc
            acc1 = jnp.zeros((8, _CCH), dtype=jnp.float32)
            for k in range(16):
                pk = (a * w0[k] + xd * w1[k] + yd * w2[k] + bb[k])
                term = v2[k] * jnp.maximum(pk, 0.0)
                if k % 2 == 0:
                    acc0 = acc0 + term
                else:
                    acc1 = acc1 + term
            logit = acc0 + acc1
            mask = 1.0 / (1.0 + jnp.exp(-logit))
            mask = jnp.where(a != 0.0, mask, 0.0)
            av = a * mask
            rid = jax.lax.broadcasted_iota(jnp.int32, (8, _CCH), 0) \
                + (row0 + ri * 8)
            cid = jax.lax.broadcasted_iota(jnp.int32, (8, _CCH), 1) + c0
            av = jnp.where(rid == cid, av + 1.0, av)
            a16_ref[rs, c0:c0 + _CCH] = av.astype(jnp.bfloat16)
            for j in range(_CCH // 128):
                sl = av[:, j * 128:(j + 1) * 128]
                rpart = sl if rpart is None else rpart + sl
            colsum_ref[0:1, c0:c0 + _CCH] = (
                colsum_ref[0:1, c0:c0 + _CCH]
                + jnp.sum(av, axis=0, keepdims=True))
        rowsum_ref[rs, :] = jnp.sum(rpart, axis=1, keepdims=True)


def _prep_kernel(rowsum_ref, colsum_ref, x_ref, w1_ref,
                 drow_ref, dcol_ref, sd_ref):
    dr = jax.lax.rsqrt(rowsum_ref[...])
    dr = jnp.where(jnp.isinf(dr), 0.0, dr)
    drow_ref[...] = jnp.broadcast_to(dr, (_N, 128))
    dc = jax.lax.rsqrt(colsum_ref[...])
    dc = jnp.where(jnp.isinf(dc), 0.0, dc)
    dcol_ref[...] = jnp.broadcast_to(dc, (_N, 128))
    sup = jnp.dot(x_ref[...].astype(jnp.bfloat16), w1_ref[...],
                  preferred_element_type=jnp.float32)
    sd_ref[...] = (sup * dc).astype(jnp.bfloat16)


def _prop1_kernel(a16_ref, sd_ref, drow_ref, dcol_ref, wl2_ref, bl2_ref,
                  w2_ref, hid_ref, s2d_ref):
    out1 = jnp.dot(a16_ref[...], sd_ref[...],
                   preferred_element_type=jnp.float32)
    out1 = out1 * drow_ref[:, 0:1]
    hid = jnp.dot(out1.astype(jnp.bfloat16), wl2_ref[...],
                  preferred_element_type=jnp.float32) + bl2_ref[...]
    hid_ref[...] = hid
    s2 = jnp.dot(hid.astype(jnp.bfloat16), w2_ref[...],
                 preferred_element_type=jnp.float32)
    s2d_ref[...] = (s2 * dcol_ref[:, 0:1]).astype(jnp.bfloat16)


def _prop2_kernel(a16_ref, s2d_ref, drow_ref, out_ref):
    o = jnp.dot(a16_ref[...], s2d_ref[...],
                preferred_element_type=jnp.float32)
    out_ref[...] = o * drow_ref[:, 0:1]


def kernel(x, adj, xdeg, ydeg, Wa1, ba1, Wa2, ba2, W1, Wl2, bl2, W2):
    f32 = jnp.float32
    v2 = (Wa2[:, 1] - Wa2[:, 0]).astype(f32)
    c = (ba2[1] - ba2[0]).astype(f32)
    params = jnp.concatenate([
        Wa1[0].astype(f32), Wa1[1].astype(f32), Wa1[2].astype(f32),
        ba1.astype(f32), v2, jnp.full((16,), c, dtype=f32),
        jnp.zeros((32,), dtype=f32)]).reshape(1, 128)

    a16, rowsum, colsum = pl.pallas_call(
        _edge_kernel,
        grid=(_N // _BR,),
        in_specs=[
            pl.BlockSpec((1, 128), lambda i: (0, 0)),
            pl.BlockSpec((_BR, _N), lambda i: (i, 0)),
            pl.BlockSpec((_BR, _N), lambda i: (i, 0)),
            pl.BlockSpec((_BR, _N), lambda i: (i, 0)),
        ],
        out_specs=[
            pl.BlockSpec((_BR, _N), lambda i: (i, 0)),
            pl.BlockSpec((_BR, 1), lambda i: (i, 0)),
            pl.BlockSpec((1, _N), lambda i: (0, 0)),
        ],
        out_shape=[
            jax.ShapeDtypeStruct((_N, _N), jnp.bfloat16),
            jax.ShapeDtypeStruct((_N, 1), f32),
            jax.ShapeDtypeStruct((1, _N), f32),
        ],
    )(params, adj, xdeg, ydeg)

    colsum_col = colsum.reshape(_N, 1)
    drow, dcol, sd = pl.pallas_call(
        _prep_kernel,
        out_shape=[
            jax.ShapeDtypeStruct((_N, 128), f32),
            jax.ShapeDtypeStruct((_N, 128), f32),
            jax.ShapeDtypeStruct((_N, 256), jnp.bfloat16),
        ],
    )(rowsum, colsum_col, x, W1.astype(jnp.bfloat16))

    w2pad = jnp.zeros((64, 128), dtype=jnp.bfloat16).at[:, :2].set(
        W2.astype(jnp.bfloat16))
    hid, s2d = pl.pallas_call(
        _prop1_kernel,
        grid=(_N // _BM,),
        in_specs=[
            pl.BlockSpec((_BM, _N), lambda i: (i, 0)),
            pl.BlockSpec((_N, 256), lambda i: (0, 0)),
            pl.BlockSpec((_BM, 128), lambda i: (i, 0)),
            pl.BlockSpec((_BM, 128), lambda i: (i, 0)),
            pl.BlockSpec((256, 64), lambda i: (0, 0)),
            pl.BlockSpec((1, 64), lambda i: (0, 0)),
            pl.BlockSpec((64, 128), lambda i: (0, 0)),
        ],
        out_specs=[
            pl.BlockSpec((_BM, 64), lambda i: (i, 0)),
            pl.BlockSpec((_BM, 128), lambda i: (i, 0)),
        ],
        out_shape=[
            jax.ShapeDtypeStruct((_N, 64), f32),
            jax.ShapeDtypeStruct((_N, 128), jnp.bfloat16),
        ],
    )(a16, sd, drow, dcol, Wl2.astype(jnp.bfloat16), bl2.reshape(1, 64),
      w2pad)

    outp = pl.pallas_call(
        _prop2_kernel,
        grid=(_N // _BM,),
        in_specs=[
            pl.BlockSpec((_BM, _N), lambda i: (i, 0)),
            pl.BlockSpec((_N, 128), lambda i: (0, 0)),
            pl.BlockSpec((_BM, 128), lambda i: (i, 0)),
        ],
        out_specs=pl.BlockSpec((_BM, 128), lambda i: (i, 0)),
        out_shape=jax.ShapeDtypeStruct((_N, 128), f32),
    )(a16, s2d, drow)

    output = outp[:, :2]
    return (output, hid, output)
